# baseline (speedup 1.0000x reference)
import jax
import jax.numpy as jnp
from jax import lax
from jax.experimental import pallas as pl
from jax.experimental.pallas import tpu as pltpu
from jax.experimental.pallas import tpu_sc as plsc

_XMINS = jnp.array([1.0728769e-07, 2.0230031, 8.1847715e-08, 136027.72, 3.0, 0.03517608, 3.4028237e+38, 2.0, 0.010238367, 1.396856e-09, 2.6865074e-10, 1.3970158e-09, 0.66699225, 0.66699225, 0.66699225, 0.75000001, 0.61141304, 0.75000001, 0.0, 7.0958774e-08, 0.0], dtype=jnp.float32)
_ALPHAS = jnp.array([1.0868737, 1.0568325, 1.082071, 27.1194, 1.0678725, 1.189603, 443107.0, 1.4543958, 1.1181921, 1.0846596, 1.0769672, 1.085144, 23.798765, 23.798765, 23.922783, 121435080.0, 3.4316596, 121435080.0, 1.08212, 1.082116, 1.08212], dtype=jnp.float32)

_CB = 48
_H = 224
_W = 224
_NSLAB = 768
_NW = 32
_SPW = _NSLAB // _NW


def _tc_body(bfac_ref, rr_ref, coin_ref, x_ref, o_ref):
    b = pl.program_id(0)
    f = bfac_ref[b]
    rr = rr_ref[0]
    cn = coin_ref[0]
    h, w = x_ref.shape[2], x_ref.shape[3]
    ih = jax.lax.broadcasted_iota(jnp.int32, (h, w), 0)
    iw = jax.lax.broadcasted_iota(jnp.int32, (h, w), 1)
    hit = jnp.where(cn == 1, iw, ih) == rr
    plane = jnp.where(hit, f, jnp.float32(1.0))
    o_ref[...] = x_ref[...] * plane[None, None]


def _tc_call(forward_input, bfac, rr, cn):
    b, c, h, w = forward_input.shape
    return pl.pallas_call(
        _tc_body,
        grid=(b, c // _CB),
        in_specs=[
            pl.BlockSpec(memory_space=pltpu.SMEM),
            pl.BlockSpec(memory_space=pltpu.SMEM),
            pl.BlockSpec(memory_space=pltpu.SMEM),
            pl.BlockSpec((1, _CB, h, w), lambda i, j: (i, j, 0, 0)),
        ],
        out_specs=pl.BlockSpec((1, _CB, h, w), lambda i, j: (i, j, 0, 0)),
        out_shape=jax.ShapeDtypeStruct((b, c, h, w), jnp.float32),
        compiler_params=pltpu.CompilerParams(vmem_limit_bytes=100 * 1024 * 1024),
    )(bfac, rr, cn, forward_input)


def _sc_body(x_hbm, fv_hbm, rc_hbm, o_hbm, buf, vbf, vrc):
    cid = lax.axis_index("c")
    sid = lax.axis_index("s")
    wid = cid * 16 + sid
    base = wid * _SPW
    pltpu.sync_copy(fv_hbm.at[wid], vbf)
    pltpu.sync_copy(rc_hbm, vrc)
    vrc16 = vrc[...]
    rr = vrc16[0]
    cn = vrc16[1]
    f = vbf[...]

    def slab(i, carry):
        idx = base + i
        pltpu.sync_copy(x_hbm.at[idx], buf)

        @pl.when(cn == 0)
        def _():
            def rowfix(j, c2):
                o = pl.multiple_of(rr * _W + j * 16, 16)
                v = buf[pl.ds(o, 16)]
                buf[pl.ds(o, 16)] = v * f
                return c2
            lax.fori_loop(0, _W // 16, rowfix, 0, unroll=7)

        @pl.when(cn == 1)
        def _():
            lane = jnp.equal(lax.iota(jnp.int32, 16), rr % 16)
            cbase = (rr // 16) * 16

            def colfix(h, c2):
                o = pl.multiple_of(h * _W + cbase, 16)
                v = buf[pl.ds(o, 16)]
                buf[pl.ds(o, 16)] = jnp.where(lane, v * f, v)
                return c2
            lax.fori_loop(0, _H, colfix, 0, unroll=8)

        pltpu.sync_copy(buf, o_hbm.at[idx])
        return carry

    lax.fori_loop(0, _SPW, slab, 0)


def _sc_call(forward_input, bfac, rr, cn):
    b, c, h, w = forward_input.shape
    x3 = forward_input.reshape(_NSLAB, h * w)
    fvec = jnp.repeat(jnp.repeat(bfac, _NW // b)[:, None], 16, axis=1)
    rc = jnp.concatenate([rr, cn, jnp.zeros((14,), jnp.int32)])
    mesh = plsc.VectorSubcoreMesh(
        core_axis_name="c", subcore_axis_name="s", num_cores=2, num_subcores=16)
    out3 = pl.kernel(
        _sc_body,
        out_type=jax.ShapeDtypeStruct((_NSLAB, h * w), jnp.float32),
        mesh=mesh,
        scratch_types=[
            pltpu.VMEM((h * w,), jnp.float32),
            pltpu.VMEM((16,), jnp.float32),
            pltpu.VMEM((16,), jnp.int32),
        ],
    )(x3, fvec, rc)
    return out3.reshape(b, c, h, w)


def _params(forward_input):
    p = 0.3
    b, c, h, w = forward_input.shape
    key = jax.random.key(42)
    k1, k2, k3, k4, k5 = jax.random.split(key, 5)
    sampled = jax.random.bernoulli(k1, p, (b,))
    rand_row = jax.random.randint(k2, (), 0, h)
    coin = jax.random.bernoulli(k3, 0.5)
    idx = jax.random.randint(k4, (), 0, _ALPHAS.shape[0])
    r = jax.random.uniform(k5, (), dtype=jnp.float32)
    alpha = _ALPHAS[idx]
    x_min = _XMINS[idx]
    rel = x_min * (1.0 - r) ** (-1.0 / (alpha - 1.0))
    bfac = jnp.where(sampled, rel, jnp.float32(1.0)).astype(jnp.float32)
    rr = rand_row.astype(jnp.int32).reshape(1)
    cn = coin.astype(jnp.int32).reshape(1)
    return bfac, rr, cn


def kernel(forward_input):
    bfac, rr, cn = _params(forward_input)
    return _sc_call(forward_input, bfac, rr, cn)


# speedup vs baseline: 1.0210x; 1.0210x over previous
import jax
import jax.numpy as jnp
from jax import lax
from jax.experimental import pallas as pl
from jax.experimental.pallas import tpu as pltpu
from jax.experimental.pallas import tpu_sc as plsc

_XMINS = jnp.array([1.0728769e-07, 2.0230031, 8.1847715e-08, 136027.72, 3.0, 0.03517608, 3.4028237e+38, 2.0, 0.010238367, 1.396856e-09, 2.6865074e-10, 1.3970158e-09, 0.66699225, 0.66699225, 0.66699225, 0.75000001, 0.61141304, 0.75000001, 0.0, 7.0958774e-08, 0.0], dtype=jnp.float32)
_ALPHAS = jnp.array([1.0868737, 1.0568325, 1.082071, 27.1194, 1.0678725, 1.189603, 443107.0, 1.4543958, 1.1181921, 1.0846596, 1.0769672, 1.085144, 23.798765, 23.798765, 23.922783, 121435080.0, 3.4316596, 121435080.0, 1.08212, 1.082116, 1.08212], dtype=jnp.float32)

_CB = 48
_H = 224
_W = 224
_NSLAB = 768
_NW = 32
_SPW = _NSLAB // _NW


def _tc_body(bfac_ref, rr_ref, coin_ref, x_ref, o_ref):
    b = pl.program_id(0)
    f = bfac_ref[b]
    rr = rr_ref[0]
    cn = coin_ref[0]
    h, w = x_ref.shape[2], x_ref.shape[3]
    ih = jax.lax.broadcasted_iota(jnp.int32, (h, w), 0)
    iw = jax.lax.broadcasted_iota(jnp.int32, (h, w), 1)
    hit = jnp.where(cn == 1, iw, ih) == rr
    plane = jnp.where(hit, f, jnp.float32(1.0))
    o_ref[...] = x_ref[...] * plane[None, None]


def _tc_call(forward_input, bfac, rr, cn):
    b, c, h, w = forward_input.shape
    return pl.pallas_call(
        _tc_body,
        grid=(b, c // _CB),
        in_specs=[
            pl.BlockSpec(memory_space=pltpu.SMEM),
            pl.BlockSpec(memory_space=pltpu.SMEM),
            pl.BlockSpec(memory_space=pltpu.SMEM),
            pl.BlockSpec((1, _CB, h, w), lambda i, j: (i, j, 0, 0)),
        ],
        out_specs=pl.BlockSpec((1, _CB, h, w), lambda i, j: (i, j, 0, 0)),
        out_shape=jax.ShapeDtypeStruct((b, c, h, w), jnp.float32),
        compiler_params=pltpu.CompilerParams(vmem_limit_bytes=100 * 1024 * 1024),
    )(bfac, rr, cn, forward_input)


def _fix_line(buf, rr, cn, f):
    @pl.when(cn == 0)
    def _():
        def rowfix(j, c2):
            o = pl.multiple_of(rr * _W + j * 16, 16)
            v = buf[pl.ds(o, 16)]
            buf[pl.ds(o, 16)] = v * f
            return c2
        lax.fori_loop(0, _W // 16, rowfix, 0, unroll=7)

    @pl.when(cn == 1)
    def _():
        lane = jnp.equal(lax.iota(jnp.int32, 16), rr % 16)
        cbase = (rr // 16) * 16

        def colfix(h, c2):
            o = pl.multiple_of(h * _W + cbase, 16)
            v = buf[pl.ds(o, 16)]
            buf[pl.ds(o, 16)] = jnp.where(lane, v * f, v)
            return c2
        lax.fori_loop(0, _H, colfix, 0, unroll=8)


def _sc_body(x_hbm, fv_hbm, rc_hbm, o_hbm, buf, vbf, vrc,
             ld0, ld1, st0, st1):
    cid = lax.axis_index("c")
    sid = lax.axis_index("s")
    wid = cid * 16 + sid
    base = wid * _SPW
    pltpu.sync_copy(fv_hbm.at[wid], vbf)
    pltpu.sync_copy(rc_hbm, vrc)
    vrc16 = vrc[...]
    rr = vrc16[0]
    cn = vrc16[1]
    f = vbf[...]

    ldsem = (ld0, ld1)
    stsem = (st0, st1)
    for i in range(_SPW):
        cur = i % 2
        nxt = 1 - cur
        if i == 0:
            pltpu.async_copy(x_hbm.at[base], buf.at[0], ldsem[0])
        if i + 1 < _SPW:
            if i >= 1:
                pltpu.make_async_copy(
                    buf.at[nxt], o_hbm.at[base + i - 1], stsem[nxt]).wait()
            pltpu.async_copy(x_hbm.at[base + i + 1], buf.at[nxt], ldsem[nxt])
        pltpu.make_async_copy(x_hbm.at[base + i], buf.at[cur], ldsem[cur]).wait()
        _fix_line(buf.at[cur], rr, cn, f)
        pltpu.async_copy(buf.at[cur], o_hbm.at[base + i], stsem[cur])
    last = (_SPW - 1) % 2
    pltpu.make_async_copy(
        buf.at[1 - last], o_hbm.at[base + _SPW - 2], stsem[1 - last]).wait()
    pltpu.make_async_copy(
        buf.at[last], o_hbm.at[base + _SPW - 1], stsem[last]).wait()


def _sc_call(forward_input, bfac, rr, cn):
    b, c, h, w = forward_input.shape
    x3 = forward_input.reshape(_NSLAB, h * w)
    fvec = jnp.repeat(jnp.repeat(bfac, _NW // b)[:, None], 16, axis=1)
    rc = jnp.concatenate([rr, cn, jnp.zeros((14,), jnp.int32)])
    mesh = plsc.VectorSubcoreMesh(
        core_axis_name="c", subcore_axis_name="s", num_cores=2, num_subcores=16)
    out3 = pl.kernel(
        _sc_body,
        out_type=jax.ShapeDtypeStruct((_NSLAB, h * w), jnp.float32),
        mesh=mesh,
        scratch_types=[
            pltpu.VMEM((2, h * w), jnp.float32),
            pltpu.VMEM((16,), jnp.float32),
            pltpu.VMEM((16,), jnp.int32),
            pltpu.SemaphoreType.DMA,
            pltpu.SemaphoreType.DMA,
            pltpu.SemaphoreType.DMA,
            pltpu.SemaphoreType.DMA,
        ],
    )(x3, fvec, rc)
    return out3.reshape(b, c, h, w)


def _params(forward_input):
    p = 0.3
    b, c, h, w = forward_input.shape
    key = jax.random.key(42)
    k1, k2, k3, k4, k5 = jax.random.split(key, 5)
    sampled = jax.random.bernoulli(k1, p, (b,))
    rand_row = jax.random.randint(k2, (), 0, h)
    coin = jax.random.bernoulli(k3, 0.5)
    idx = jax.random.randint(k4, (), 0, _ALPHAS.shape[0])
    r = jax.random.uniform(k5, (), dtype=jnp.float32)
    alpha = _ALPHAS[idx]
    x_min = _XMINS[idx]
    rel = x_min * (1.0 - r) ** (-1.0 / (alpha - 1.0))
    bfac = jnp.where(sampled, rel, jnp.float32(1.0)).astype(jnp.float32)
    rr = rand_row.astype(jnp.int32).reshape(1)
    cn = coin.astype(jnp.int32).reshape(1)
    return bfac, rr, cn


def kernel(forward_input):
    bfac, rr, cn = _params(forward_input)
    return _sc_call(forward_input, bfac, rr, cn)


# speedup vs baseline: 3.0396x; 2.9771x over previous
import jax
import jax.numpy as jnp
from jax import lax
from jax.experimental import pallas as pl
from jax.experimental.pallas import tpu as pltpu
from jax.experimental.pallas import tpu_sc as plsc

_XMINS = jnp.array([1.0728769e-07, 2.0230031, 8.1847715e-08, 136027.72, 3.0, 0.03517608, 3.4028237e+38, 2.0, 0.010238367, 1.396856e-09, 2.6865074e-10, 1.3970158e-09, 0.66699225, 0.66699225, 0.66699225, 0.75000001, 0.61141304, 0.75000001, 0.0, 7.0958774e-08, 0.0], dtype=jnp.float32)
_ALPHAS = jnp.array([1.0868737, 1.0568325, 1.082071, 27.1194, 1.0678725, 1.189603, 443107.0, 1.4543958, 1.1181921, 1.0846596, 1.0769672, 1.085144, 23.798765, 23.798765, 23.922783, 121435080.0, 3.4316596, 121435080.0, 1.08212, 1.082116, 1.08212], dtype=jnp.float32)

_CB = 24
_H = 224
_W = 224
_NSLAB = 768
_NW = 32
_SPW = _NSLAB // _NW


def _tc_body(bfac_ref, rr_ref, coin_ref, x_ref, o_ref):
    b = pl.program_id(0)
    f = bfac_ref[b]
    rr = rr_ref[0]
    cn = coin_ref[0]
    h, w = x_ref.shape[2], x_ref.shape[3]
    ih = jax.lax.broadcasted_iota(jnp.int32, (h, w), 0)
    iw = jax.lax.broadcasted_iota(jnp.int32, (h, w), 1)
    hit = jnp.where(cn == 1, iw, ih) == rr
    plane = jnp.where(hit, f, jnp.float32(1.0))
    o_ref[...] = x_ref[...] * plane[None, None]


def _tc_call(forward_input, bfac, rr, cn):
    b, c, h, w = forward_input.shape
    return pl.pallas_call(
        _tc_body,
        grid=(b, c // _CB),
        in_specs=[
            pl.BlockSpec(memory_space=pltpu.SMEM),
            pl.BlockSpec(memory_space=pltpu.SMEM),
            pl.BlockSpec(memory_space=pltpu.SMEM),
            pl.BlockSpec((1, _CB, h, w), lambda i, j: (i, j, 0, 0)),
        ],
        out_specs=pl.BlockSpec((1, _CB, h, w), lambda i, j: (i, j, 0, 0)),
        out_shape=jax.ShapeDtypeStruct((b, c, h, w), jnp.float32),
        compiler_params=pltpu.CompilerParams(vmem_limit_bytes=100 * 1024 * 1024),
    )(bfac, rr, cn, forward_input)


def _fix_line(buf, rr, cn, f):
    @pl.when(cn == 0)
    def _():
        def rowfix(j, c2):
            o = pl.multiple_of(rr * _W + j * 16, 16)
            v = buf[pl.ds(o, 16)]
            buf[pl.ds(o, 16)] = v * f
            return c2
        lax.fori_loop(0, _W // 16, rowfix, 0, unroll=7)

    @pl.when(cn == 1)
    def _():
        lane = jnp.equal(lax.iota(jnp.int32, 16), rr % 16)
        cbase = (rr // 16) * 16

        def colfix(h, c2):
            o = pl.multiple_of(h * _W + cbase, 16)
            v = buf[pl.ds(o, 16)]
            buf[pl.ds(o, 16)] = jnp.where(lane, v * f, v)
            return c2
        lax.fori_loop(0, _H, colfix, 0, unroll=8)


def _sc_body(x_hbm, fv_hbm, rc_hbm, o_hbm, buf, vbf, vrc,
             ld0, ld1, st0, st1):
    cid = lax.axis_index("c")
    sid = lax.axis_index("s")
    wid = cid * 16 + sid
    base = wid * _SPW
    pltpu.sync_copy(fv_hbm.at[wid], vbf)
    pltpu.sync_copy(rc_hbm, vrc)
    vrc16 = vrc[...]
    rr = vrc16[0]
    cn = vrc16[1]
    f = vbf[...]

    ldsem = (ld0, ld1)
    stsem = (st0, st1)
    for i in range(_SPW):
        cur = i % 2
        nxt = 1 - cur
        if i == 0:
            pltpu.async_copy(x_hbm.at[base], buf.at[0], ldsem[0])
        if i + 1 < _SPW:
            if i >= 1:
                pltpu.make_async_copy(
                    buf.at[nxt], o_hbm.at[base + i - 1], stsem[nxt]).wait()
            pltpu.async_copy(x_hbm.at[base + i + 1], buf.at[nxt], ldsem[nxt])
        pltpu.make_async_copy(x_hbm.at[base + i], buf.at[cur], ldsem[cur]).wait()
        _fix_line(buf.at[cur], rr, cn, f)
        pltpu.async_copy(buf.at[cur], o_hbm.at[base + i], stsem[cur])
    last = (_SPW - 1) % 2
    pltpu.make_async_copy(
        buf.at[1 - last], o_hbm.at[base + _SPW - 2], stsem[1 - last]).wait()
    pltpu.make_async_copy(
        buf.at[last], o_hbm.at[base + _SPW - 1], stsem[last]).wait()


def _sc_call(forward_input, bfac, rr, cn):
    b, c, h, w = forward_input.shape
    x3 = forward_input.reshape(_NSLAB, h * w)
    fvec = jnp.repeat(jnp.repeat(bfac, _NW // b)[:, None], 16, axis=1)
    rc = jnp.concatenate([rr, cn, jnp.zeros((14,), jnp.int32)])
    mesh = plsc.VectorSubcoreMesh(
        core_axis_name="c", subcore_axis_name="s", num_cores=2, num_subcores=16)
    out3 = pl.kernel(
        _sc_body,
        out_type=jax.ShapeDtypeStruct((_NSLAB, h * w), jnp.float32),
        mesh=mesh,
        scratch_types=[
            pltpu.VMEM((2, h * w), jnp.float32),
            pltpu.VMEM((16,), jnp.float32),
            pltpu.VMEM((16,), jnp.int32),
            pltpu.SemaphoreType.DMA,
            pltpu.SemaphoreType.DMA,
            pltpu.SemaphoreType.DMA,
            pltpu.SemaphoreType.DMA,
        ],
    )(x3, fvec, rc)
    return out3.reshape(b, c, h, w)


def _params(forward_input):
    p = 0.3
    b, c, h, w = forward_input.shape
    key = jax.random.key(42)
    k1, k2, k3, k4, k5 = jax.random.split(key, 5)
    sampled = jax.random.bernoulli(k1, p, (b,))
    rand_row = jax.random.randint(k2, (), 0, h)
    coin = jax.random.bernoulli(k3, 0.5)
    idx = jax.random.randint(k4, (), 0, _ALPHAS.shape[0])
    r = jax.random.uniform(k5, (), dtype=jnp.float32)
    alpha = _ALPHAS[idx]
    x_min = _XMINS[idx]
    rel = x_min * (1.0 - r) ** (-1.0 / (alpha - 1.0))
    bfac = jnp.where(sampled, rel, jnp.float32(1.0)).astype(jnp.float32)
    rr = rand_row.astype(jnp.int32).reshape(1)
    cn = coin.astype(jnp.int32).reshape(1)
    return bfac, rr, cn


def kernel(forward_input):
    bfac, rr, cn = _params(forward_input)
    return _tc_call(forward_input, bfac, rr, cn)


# speedup vs baseline: 3.0637x; 1.0079x over previous
import jax
import jax.numpy as jnp
from jax import lax
from jax.experimental import pallas as pl
from jax.experimental.pallas import tpu as pltpu
from jax.experimental.pallas import tpu_sc as plsc

_XMINS = jnp.array([1.0728769e-07, 2.0230031, 8.1847715e-08, 136027.72, 3.0, 0.03517608, 3.4028237e+38, 2.0, 0.010238367, 1.396856e-09, 2.6865074e-10, 1.3970158e-09, 0.66699225, 0.66699225, 0.66699225, 0.75000001, 0.61141304, 0.75000001, 0.0, 7.0958774e-08, 0.0], dtype=jnp.float32)
_ALPHAS = jnp.array([1.0868737, 1.0568325, 1.082071, 27.1194, 1.0678725, 1.189603, 443107.0, 1.4543958, 1.1181921, 1.0846596, 1.0769672, 1.085144, 23.798765, 23.798765, 23.922783, 121435080.0, 3.4316596, 121435080.0, 1.08212, 1.082116, 1.08212], dtype=jnp.float32)

_CB = 48
_H = 224
_W = 224
_NSLAB = 768
_NW = 32
_SPW = _NSLAB // _NW


def _tc_body(bfac_ref, rr_ref, coin_ref, x_ref, o_ref):
    b = pl.program_id(0)
    f = bfac_ref[b]
    rr = rr_ref[0]
    cn = coin_ref[0]
    h, w = x_ref.shape[2], x_ref.shape[3]
    ih = jax.lax.broadcasted_iota(jnp.int32, (h, w), 0)
    iw = jax.lax.broadcasted_iota(jnp.int32, (h, w), 1)
    hit = jnp.where(cn == 1, iw, ih) == rr
    plane = jnp.where(hit, f, jnp.float32(1.0))
    o_ref[...] = x_ref[...] * plane[None, None]


def _tc_call(forward_input, bfac, rr, cn):
    b, c, h, w = forward_input.shape
    return pl.pallas_call(
        _tc_body,
        grid=(b, c // _CB),
        in_specs=[
            pl.BlockSpec(memory_space=pltpu.SMEM),
            pl.BlockSpec(memory_space=pltpu.SMEM),
            pl.BlockSpec(memory_space=pltpu.SMEM),
            pl.BlockSpec((1, _CB, h, w), lambda i, j: (i, j, 0, 0)),
        ],
        out_specs=pl.BlockSpec((1, _CB, h, w), lambda i, j: (i, j, 0, 0)),
        out_shape=jax.ShapeDtypeStruct((b, c, h, w), jnp.float32),
        compiler_params=pltpu.CompilerParams(vmem_limit_bytes=100 * 1024 * 1024, dimension_semantics=("parallel", "parallel")),
    )(bfac, rr, cn, forward_input)


def _fix_line(buf, rr, cn, f):
    @pl.when(cn == 0)
    def _():
        def rowfix(j, c2):
            o = pl.multiple_of(rr * _W + j * 16, 16)
            v = buf[pl.ds(o, 16)]
            buf[pl.ds(o, 16)] = v * f
            return c2
        lax.fori_loop(0, _W // 16, rowfix, 0, unroll=7)

    @pl.when(cn == 1)
    def _():
        lane = jnp.equal(lax.iota(jnp.int32, 16), rr % 16)
        cbase = (rr // 16) * 16

        def colfix(h, c2):
            o = pl.multiple_of(h * _W + cbase, 16)
            v = buf[pl.ds(o, 16)]
            buf[pl.ds(o, 16)] = jnp.where(lane, v * f, v)
            return c2
        lax.fori_loop(0, _H, colfix, 0, unroll=8)


def _sc_body(x_hbm, fv_hbm, rc_hbm, o_hbm, buf, vbf, vrc,
             ld0, ld1, st0, st1):
    cid = lax.axis_index("c")
    sid = lax.axis_index("s")
    wid = cid * 16 + sid
    base = wid * _SPW
    pltpu.sync_copy(fv_hbm.at[wid], vbf)
    pltpu.sync_copy(rc_hbm, vrc)
    vrc16 = vrc[...]
    rr = vrc16[0]
    cn = vrc16[1]
    f = vbf[...]

    ldsem = (ld0, ld1)
    stsem = (st0, st1)
    for i in range(_SPW):
        cur = i % 2
        nxt = 1 - cur
        if i == 0:
            pltpu.async_copy(x_hbm.at[base], buf.at[0], ldsem[0])
        if i + 1 < _SPW:
            if i >= 1:
                pltpu.make_async_copy(
                    buf.at[nxt], o_hbm.at[base + i - 1], stsem[nxt]).wait()
            pltpu.async_copy(x_hbm.at[base + i + 1], buf.at[nxt], ldsem[nxt])
        pltpu.make_async_copy(x_hbm.at[base + i], buf.at[cur], ldsem[cur]).wait()
        _fix_line(buf.at[cur], rr, cn, f)
        pltpu.async_copy(buf.at[cur], o_hbm.at[base + i], stsem[cur])
    last = (_SPW - 1) % 2
    pltpu.make_async_copy(
        buf.at[1 - last], o_hbm.at[base + _SPW - 2], stsem[1 - last]).wait()
    pltpu.make_async_copy(
        buf.at[last], o_hbm.at[base + _SPW - 1], stsem[last]).wait()


def _sc_call(forward_input, bfac, rr, cn):
    b, c, h, w = forward_input.shape
    x3 = forward_input.reshape(_NSLAB, h * w)
    fvec = jnp.repeat(jnp.repeat(bfac, _NW // b)[:, None], 16, axis=1)
    rc = jnp.concatenate([rr, cn, jnp.zeros((14,), jnp.int32)])
    mesh = plsc.VectorSubcoreMesh(
        core_axis_name="c", subcore_axis_name="s", num_cores=2, num_subcores=16)
    out3 = pl.kernel(
        _sc_body,
        out_type=jax.ShapeDtypeStruct((_NSLAB, h * w), jnp.float32),
        mesh=mesh,
        scratch_types=[
            pltpu.VMEM((2, h * w), jnp.float32),
            pltpu.VMEM((16,), jnp.float32),
            pltpu.VMEM((16,), jnp.int32),
            pltpu.SemaphoreType.DMA,
            pltpu.SemaphoreType.DMA,
            pltpu.SemaphoreType.DMA,
            pltpu.SemaphoreType.DMA,
        ],
    )(x3, fvec, rc)
    return out3.reshape(b, c, h, w)


def _params(forward_input):
    p = 0.3
    b, c, h, w = forward_input.shape
    key = jax.random.key(42)
    k1, k2, k3, k4, k5 = jax.random.split(key, 5)
    sampled = jax.random.bernoulli(k1, p, (b,))
    rand_row = jax.random.randint(k2, (), 0, h)
    coin = jax.random.bernoulli(k3, 0.5)
    idx = jax.random.randint(k4, (), 0, _ALPHAS.shape[0])
    r = jax.random.uniform(k5, (), dtype=jnp.float32)
    alpha = _ALPHAS[idx]
    x_min = _XMINS[idx]
    rel = x_min * (1.0 - r) ** (-1.0 / (alpha - 1.0))
    bfac = jnp.where(sampled, rel, jnp.float32(1.0)).astype(jnp.float32)
    rr = rand_row.astype(jnp.int32).reshape(1)
    cn = coin.astype(jnp.int32).reshape(1)
    return bfac, rr, cn


def kernel(forward_input):
    bfac, rr, cn = _params(forward_input)
    return _tc_call(forward_input, bfac, rr, cn)
